# baseline (speedup 1.0000x reference)
import jax
import jax.numpy as jnp
from jax import lax
from jax.experimental import pallas as pl
from jax.experimental.pallas import tpu as pltpu
from jax.experimental.pallas import tpu_sc as plsc

VOCAB = 100000
D = 128
B = 16384
NC = 2
NS = 16
NW = NC * NS
BW = B // NW
C = 128
NCHUNK = BW // C


def _dot_body(i_hbm, j_hbm, wi_hbm, wj_hbm, out_hbm,
              idx_i, idx_j, ri0, ri1, rj0, rj1, out_v, sem0, sem1):
    wid = lax.axis_index("s") * NC + lax.axis_index("c")
    base = wid * BW

    pltpu.sync_copy(i_hbm.at[pl.ds(base, BW)], idx_i)
    pltpu.sync_copy(j_hbm.at[pl.ds(base, BW)], idx_j)

    ri = (ri0, ri1)
    rj = (rj0, rj1)
    sems = (sem0, sem1)

    def fire(c):
        b = c % 2
        return (
            pltpu.async_copy(wi_hbm.at[idx_i.at[pl.ds(c * C, C)]], ri[b],
                             sems[b]),
            pltpu.async_copy(wj_hbm.at[idx_j.at[pl.ds(c * C, C)]], rj[b],
                             sems[b]),
        )

    inflight = fire(0)

    def zero_body(g, _):
        out_v[pl.ds(g * 16, 16)] = jnp.zeros((16,), jnp.float32)
        return 0

    lax.fori_loop(0, BW // 16, zero_body, 0)

    for c in range(NCHUNK):
        nxt = fire(c + 1) if c + 1 < NCHUNK else None
        ci, cj = inflight
        ci.wait()
        cj.wait()
        rows_i = ri[c % 2]
        rows_j = rj[c % 2]

        def row_body(r, _, c=c, rows_i=rows_i, rows_j=rows_j):
            acc0 = rows_i[r, pl.ds(0, 16)] * rows_j[r, pl.ds(0, 16)]
            acc1 = rows_i[r, pl.ds(16, 16)] * rows_j[r, pl.ds(16, 16)]
            for k in range(2, 8, 2):
                acc0 = acc0 + (rows_i[r, pl.ds(16 * k, 16)]
                               * rows_j[r, pl.ds(16 * k, 16)])
                acc1 = acc1 + (rows_i[r, pl.ds(16 * (k + 1), 16)]
                               * rows_j[r, pl.ds(16 * (k + 1), 16)])
            ridx = jnp.broadcast_to(c * C + r, (16,)).astype(jnp.int32)
            plsc.addupdate_scatter(out_v, [ridx], acc0 + acc1)
            return 0

        lax.fori_loop(0, C, row_body, 0)
        inflight = nxt

    pltpu.sync_copy(out_v, out_hbm.at[pl.ds(base, BW)])


def _bias_body(i_hbm, j_hbm, bi_hbm, bj_hbm, dot_hbm, out_hbm,
               idx_i, idx_j, bi_v, bj_v, out_v, sem_bias):
    wid = lax.axis_index("s") * NC + lax.axis_index("c")
    base = wid * BW

    pltpu.sync_copy(i_hbm.at[pl.ds(base, BW)], idx_i)
    pltpu.sync_copy(j_hbm.at[pl.ds(base, BW)], idx_j)

    cb_i = pltpu.async_copy(bi_hbm.at[idx_i], bi_v, sem_bias)
    cb_j = pltpu.async_copy(bj_hbm.at[idx_j], bj_v, sem_bias)
    pltpu.sync_copy(dot_hbm.at[pl.ds(base, BW)], out_v)
    cb_i.wait()
    cb_j.wait()

    def add_body(g, _):
        sl = pl.ds(g * 16, 16)
        out_v[sl] = out_v[sl] + bi_v[sl] + bj_v[sl]
        return 0

    lax.fori_loop(0, BW // 16, add_body, 0)

    pltpu.sync_copy(out_v, out_hbm.at[pl.ds(base, BW)])


@jax.jit
def _glove(i, j, wi, wj, bi, bj):
    mesh = plsc.VectorSubcoreMesh(core_axis_name="c", subcore_axis_name="s",
                                  num_cores=NC, num_subcores=NS)
    params = pltpu.CompilerParams(needs_layout_passes=False)

    dot_out = pl.kernel(
        _dot_body,
        out_type=jax.ShapeDtypeStruct((B,), jnp.float32),
        mesh=mesh,
        compiler_params=params,
        scratch_types=[
            pltpu.VMEM((BW,), jnp.int32),
            pltpu.VMEM((BW,), jnp.int32),
            pltpu.VMEM((C, D), jnp.float32),
            pltpu.VMEM((C, D), jnp.float32),
            pltpu.VMEM((C, D), jnp.float32),
            pltpu.VMEM((C, D), jnp.float32),
            pltpu.VMEM((BW,), jnp.float32),
            pltpu.SemaphoreType.DMA,
            pltpu.SemaphoreType.DMA,
        ],
    )(i, j, wi, wj)

    bi1 = bi.reshape(VOCAB)
    bj1 = bj.reshape(VOCAB)

    out = pl.kernel(
        _bias_body,
        out_type=jax.ShapeDtypeStruct((B,), jnp.float32),
        mesh=mesh,
        compiler_params=params,
        scratch_types=[
            pltpu.VMEM((BW,), jnp.int32),
            pltpu.VMEM((BW,), jnp.int32),
            pltpu.VMEM((BW,), jnp.float32),
            pltpu.VMEM((BW,), jnp.float32),
            pltpu.VMEM((BW,), jnp.float32),
            pltpu.SemaphoreType.DMA,
        ],
    )(i, j, bi1, bj1, dot_out)
    return out


def kernel(i, j, wi, wj, bi, bj):
    return _glove(i, j, wi, wj, bi, bj)


# speedup vs baseline: 1.3899x; 1.3899x over previous
import jax
import jax.numpy as jnp
from jax import lax
from jax.experimental import pallas as pl
from jax.experimental.pallas import tpu as pltpu
from jax.experimental.pallas import tpu_sc as plsc

VOCAB = 100000
D = 128
B = 16384
NC = 2
NS = 16
NW = NC * NS
BW = B // NW
C = 128
NCHUNK = BW // C


def _glove_body(i_hbm, j_hbm, wi_hbm, wj_hbm, bc_hbm, out_hbm,
                idx_i, idx_j, idx_jb, ri0, ri1, rj0, rj1, bi_v, bj_v, out_v,
                sem0, sem1, sem_bias):
    wid = lax.axis_index("s") * NC + lax.axis_index("c")
    base = wid * BW

    pltpu.sync_copy(i_hbm.at[pl.ds(base, BW)], idx_i)
    pltpu.sync_copy(j_hbm.at[pl.ds(base, BW)], idx_j)

    def shift_body(g, _):
        sl = pl.ds(g * 16, 16)
        idx_jb[sl] = idx_j[sl] + VOCAB
        return 0

    lax.fori_loop(0, BW // 16, shift_body, 0)

    cb_i = pltpu.async_copy(bc_hbm.at[idx_i], bi_v, sem_bias)
    cb_j = pltpu.async_copy(bc_hbm.at[idx_jb], bj_v, sem_bias)

    ri = (ri0, ri1)
    rj = (rj0, rj1)
    sems = (sem0, sem1)

    def fire(c):
        b = c % 2
        return (
            pltpu.async_copy(wi_hbm.at[idx_i.at[pl.ds(c * C, C)]], ri[b],
                             sems[b]),
            pltpu.async_copy(wj_hbm.at[idx_j.at[pl.ds(c * C, C)]], rj[b],
                             sems[b]),
        )

    inflight = fire(0)

    cb_i.wait()
    cb_j.wait()

    def bias_body(g, _):
        sl = pl.ds(g * 16, 16)
        out_v[sl] = bi_v[sl] + bj_v[sl]
        return 0

    lax.fori_loop(0, BW // 16, bias_body, 0)

    for c in range(NCHUNK):
        nxt = fire(c + 1) if c + 1 < NCHUNK else None
        ci, cj = inflight
        ci.wait()
        cj.wait()
        rows_i = ri[c % 2]
        rows_j = rj[c % 2]

        def row_body(r, _, c=c, rows_i=rows_i, rows_j=rows_j):
            acc0 = rows_i[r, pl.ds(0, 16)] * rows_j[r, pl.ds(0, 16)]
            acc1 = rows_i[r, pl.ds(16, 16)] * rows_j[r, pl.ds(16, 16)]
            for k in range(2, 8, 2):
                acc0 = acc0 + (rows_i[r, pl.ds(16 * k, 16)]
                               * rows_j[r, pl.ds(16 * k, 16)])
                acc1 = acc1 + (rows_i[r, pl.ds(16 * (k + 1), 16)]
                               * rows_j[r, pl.ds(16 * (k + 1), 16)])
            ridx = jnp.broadcast_to(c * C + r, (16,)).astype(jnp.int32)
            plsc.addupdate_scatter(out_v, [ridx], acc0 + acc1)
            return 0

        lax.fori_loop(0, C, row_body, 0)
        inflight = nxt

    pltpu.sync_copy(out_v, out_hbm.at[pl.ds(base, BW)])


@jax.jit
def _glove(i, j, wi, wj, bc):
    mesh = plsc.VectorSubcoreMesh(core_axis_name="c", subcore_axis_name="s",
                                  num_cores=NC, num_subcores=NS)
    run = pl.kernel(
        _glove_body,
        out_type=jax.ShapeDtypeStruct((B,), jnp.float32),
        mesh=mesh,
        compiler_params=pltpu.CompilerParams(needs_layout_passes=False),
        scratch_types=[
            pltpu.VMEM((BW,), jnp.int32),
            pltpu.VMEM((BW,), jnp.int32),
            pltpu.VMEM((BW,), jnp.int32),
            pltpu.VMEM((C, D), jnp.float32),
            pltpu.VMEM((C, D), jnp.float32),
            pltpu.VMEM((C, D), jnp.float32),
            pltpu.VMEM((C, D), jnp.float32),
            pltpu.VMEM((BW,), jnp.float32),
            pltpu.VMEM((BW,), jnp.float32),
            pltpu.VMEM((BW,), jnp.float32),
            pltpu.SemaphoreType.DMA,
            pltpu.SemaphoreType.DMA,
            pltpu.SemaphoreType.DMA,
        ],
    )
    return run(i, j, wi, wj, bc)


def kernel(i, j, wi, wj, bi, bj):
    bc = jnp.concatenate([bi, bj], axis=0).reshape(2 * VOCAB)
    return _glove(i, j, wi, wj, bc)


# speedup vs baseline: 1.5484x; 1.1140x over previous
import jax
import jax.numpy as jnp
from jax import lax
from jax.experimental import pallas as pl
from jax.experimental.pallas import tpu as pltpu
from jax.experimental.pallas import tpu_sc as plsc

VOCAB = 100000
D = 128
B = 16384
NC = 2
NS = 16
NW = NC * NS
BW = B // NW
C = 128
NCHUNK = BW // C


def _glove_body(i_hbm, j_hbm, wi_hbm, wj_hbm, bi_hbm, bj_hbm, out_hbm,
                idx_i, idx_j, ri0, ri1, rj0, rj1, bi_v, bj_v, out_v,
                sem0, sem1, sem_bias):
    wid = lax.axis_index("s") * NC + lax.axis_index("c")
    base = wid * BW

    pltpu.sync_copy(i_hbm.at[pl.ds(base, BW)], idx_i)
    pltpu.sync_copy(j_hbm.at[pl.ds(base, BW)], idx_j)

    cb_i = pltpu.async_copy(bi_hbm.at[idx_i], bi_v, sem_bias)
    cb_j = pltpu.async_copy(bj_hbm.at[idx_j], bj_v, sem_bias)

    ri = (ri0, ri1)
    rj = (rj0, rj1)
    sems = (sem0, sem1)

    H = C // 2

    def fire(c):
        b = c % 2
        return (
            pltpu.async_copy(wi_hbm.at[idx_i.at[pl.ds(c * C, H)]],
                             ri[b].at[pl.ds(0, H)], sems[b]),
            pltpu.async_copy(wi_hbm.at[idx_i.at[pl.ds(c * C + H, H)]],
                             ri[b].at[pl.ds(H, H)], sems[b]),
            pltpu.async_copy(wj_hbm.at[idx_j.at[pl.ds(c * C, H)]],
                             rj[b].at[pl.ds(0, H)], sems[b]),
            pltpu.async_copy(wj_hbm.at[idx_j.at[pl.ds(c * C + H, H)]],
                             rj[b].at[pl.ds(H, H)], sems[b]),
        )

    inflight = fire(0)

    cb_i.wait()
    cb_j.wait()

    def bias_body(g, _):
        sl = pl.ds(g * 16, 16)
        out_v[sl] = bi_v[sl] + bj_v[sl]
        return 0

    lax.fori_loop(0, BW // 16, bias_body, 0)

    for c in range(NCHUNK):
        nxt = fire(c + 1) if c + 1 < NCHUNK else None
        for cp in inflight:
            cp.wait()
        rows_i = ri[c % 2]
        rows_j = rj[c % 2]

        def row_body(r, _, c=c, rows_i=rows_i, rows_j=rows_j):
            acc0 = rows_i[r, pl.ds(0, 16)] * rows_j[r, pl.ds(0, 16)]
            acc1 = rows_i[r, pl.ds(16, 16)] * rows_j[r, pl.ds(16, 16)]
            for k in range(2, 8, 2):
                acc0 = acc0 + (rows_i[r, pl.ds(16 * k, 16)]
                               * rows_j[r, pl.ds(16 * k, 16)])
                acc1 = acc1 + (rows_i[r, pl.ds(16 * (k + 1), 16)]
                               * rows_j[r, pl.ds(16 * (k + 1), 16)])
            ridx = jnp.broadcast_to(c * C + r, (16,)).astype(jnp.int32)
            plsc.addupdate_scatter(out_v, [ridx], acc0 + acc1)
            return 0

        lax.fori_loop(0, C, row_body, 0)
        inflight = nxt

    pltpu.sync_copy(out_v, out_hbm.at[pl.ds(base, BW)])


@jax.jit
def _glove(i, j, wi, wj, bi, bj):
    mesh = plsc.VectorSubcoreMesh(core_axis_name="c", subcore_axis_name="s",
                                  num_cores=NC, num_subcores=NS)
    run = pl.kernel(
        _glove_body,
        out_type=jax.ShapeDtypeStruct((B,), jnp.float32),
        mesh=mesh,
        compiler_params=pltpu.CompilerParams(needs_layout_passes=False),
        scratch_types=[
            pltpu.VMEM((BW,), jnp.int32),
            pltpu.VMEM((BW,), jnp.int32),
            pltpu.VMEM((C, D), jnp.float32),
            pltpu.VMEM((C, D), jnp.float32),
            pltpu.VMEM((C, D), jnp.float32),
            pltpu.VMEM((C, D), jnp.float32),
            pltpu.VMEM((BW,), jnp.float32),
            pltpu.VMEM((BW,), jnp.float32),
            pltpu.VMEM((BW,), jnp.float32),
            pltpu.SemaphoreType.DMA,
            pltpu.SemaphoreType.DMA,
            pltpu.SemaphoreType.DMA,
        ],
    )
    return run(i, j, wi, wj, bi, bj)


def kernel(i, j, wi, wj, bi, bj):
    return _glove(i, j, wi, wj, bi.reshape(VOCAB), bj.reshape(VOCAB))


# speedup vs baseline: 1.5763x; 1.0180x over previous
import jax
import jax.numpy as jnp
from jax import lax
from jax.experimental import pallas as pl
from jax.experimental.pallas import tpu as pltpu
from jax.experimental.pallas import tpu_sc as plsc

VOCAB = 100000
D = 128
B = 16384
NC = 2
NS = 16
NW = NC * NS
BW = B // NW
C = 128
NCHUNK = BW // C


def _glove_body(i_hbm, j_hbm, wi_hbm, wj_hbm, bi_hbm, bj_hbm, out_hbm,
                idx_i, idx_j, ri0, ri1, rj0, rj1, bi_v, bj_v, out_v,
                sem0, sem1, sem_bias):
    wid = lax.axis_index("s") * NC + lax.axis_index("c")
    base = wid * BW

    ca = pltpu.async_copy(i_hbm.at[pl.ds(base, BW)], idx_i, sem0)
    cb = pltpu.async_copy(j_hbm.at[pl.ds(base, BW)], idx_j, sem1)
    ca.wait()
    cb.wait()

    ri = (ri0, ri1)
    rj = (rj0, rj1)
    sems = (sem0, sem1)

    def fire(c):
        b = c % 2
        return (
            pltpu.async_copy(wi_hbm.at[idx_i.at[pl.ds(c * C, C)]], ri[b],
                             sems[b]),
            pltpu.async_copy(wj_hbm.at[idx_j.at[pl.ds(c * C, C)]], rj[b],
                             sems[b]),
        )

    inflight = fire(0)

    cb_i = pltpu.async_copy(bi_hbm.at[idx_i], bi_v, sem_bias)
    cb_j = pltpu.async_copy(bj_hbm.at[idx_j], bj_v, sem_bias)

    cb_i.wait()
    cb_j.wait()

    def bias_body(g, _):
        sl = pl.ds(g * 16, 16)
        out_v[sl] = bi_v[sl] + bj_v[sl]
        return 0

    lax.fori_loop(0, BW // 16, bias_body, 0)

    for c in range(NCHUNK):
        nxt = fire(c + 1) if c + 1 < NCHUNK else None
        ci, cj = inflight
        ci.wait()
        cj.wait()
        rows_i = ri[c % 2]
        rows_j = rj[c % 2]

        def row_body(r, _, c=c, rows_i=rows_i, rows_j=rows_j):
            acc0 = rows_i[r, pl.ds(0, 16)] * rows_j[r, pl.ds(0, 16)]
            acc1 = rows_i[r, pl.ds(16, 16)] * rows_j[r, pl.ds(16, 16)]
            for k in range(2, 8, 2):
                acc0 = acc0 + (rows_i[r, pl.ds(16 * k, 16)]
                               * rows_j[r, pl.ds(16 * k, 16)])
                acc1 = acc1 + (rows_i[r, pl.ds(16 * (k + 1), 16)]
                               * rows_j[r, pl.ds(16 * (k + 1), 16)])
            ridx = jnp.broadcast_to(c * C + r, (16,)).astype(jnp.int32)
            plsc.addupdate_scatter(out_v, [ridx], acc0 + acc1)
            return 0

        lax.fori_loop(0, C, row_body, 0)
        inflight = nxt

    pltpu.sync_copy(out_v, out_hbm.at[pl.ds(base, BW)])


@jax.jit
def _glove(i, j, wi, wj, bi, bj):
    mesh = plsc.VectorSubcoreMesh(core_axis_name="c", subcore_axis_name="s",
                                  num_cores=NC, num_subcores=NS)
    run = pl.kernel(
        _glove_body,
        out_type=jax.ShapeDtypeStruct((B,), jnp.float32),
        mesh=mesh,
        compiler_params=pltpu.CompilerParams(needs_layout_passes=False),
        scratch_types=[
            pltpu.VMEM((BW,), jnp.int32),
            pltpu.VMEM((BW,), jnp.int32),
            pltpu.VMEM((C, D), jnp.float32),
            pltpu.VMEM((C, D), jnp.float32),
            pltpu.VMEM((C, D), jnp.float32),
            pltpu.VMEM((C, D), jnp.float32),
            pltpu.VMEM((BW,), jnp.float32),
            pltpu.VMEM((BW,), jnp.float32),
            pltpu.VMEM((BW,), jnp.float32),
            pltpu.SemaphoreType.DMA,
            pltpu.SemaphoreType.DMA,
            pltpu.SemaphoreType.DMA,
        ],
    )
    return run(i, j, wi, wj, bi, bj)


def kernel(i, j, wi, wj, bi, bj):
    return _glove(i, j, wi, wj, bi.reshape(VOCAB), bj.reshape(VOCAB))
